# baseline (speedup 1.0000x reference)
import functools

import jax
import jax.numpy as jnp
from jax import lax
from jax.experimental import pallas as pl
from jax.experimental.pallas import tpu as pltpu
from jax.experimental.pallas import tpu_sc as plsc

B, S, D = 4, 8192, 768
DINUC_DIM = D // 4
EPS = 1e-12
TOK = B * S
NC, NS = 2, 16
NW = NC * NS
BPW = TOK // NW
CH = 64
NCH = BPW // CH


def _prep_body(ids_ref, tt_ref, dt_ref, w_ref, b_ref, g_ref, be_ref,
               lut_ref, c_ref):
    w_top = w_ref[:D, :]
    w_bot = w_ref[D:, :]
    tf = jnp.dot(tt_ref[...], w_top, preferred_element_type=jnp.float32)
    df = jnp.dot(dt_ref[...], w_bot, preferred_element_type=jnp.float32)
    df32 = jnp.concatenate([df, jnp.zeros((16, D), jnp.float32)], axis=0)
    x = tf[:, None, :] + df32[None, :, :] + b_ref[...][None, :, :]
    mean = jnp.mean(x, axis=-1, keepdims=True)
    var = jnp.mean((x - mean) ** 2, axis=-1, keepdims=True)
    lut_ref[...] = ((x - mean) * lax.rsqrt(var + EPS)
                    * g_ref[...][None, :, :] + be_ref[...][None, :, :])

    first = ids_ref[...]
    second = jnp.concatenate(
        [first[:, 1:], jnp.zeros((B, 1), jnp.int32)], axis=1)
    valid = ((first >= 4) & (first <= 7) & (second >= 4) & (second <= 7))
    did = jnp.where(valid, (first - 4) * 4 + (second - 4), 0)
    col = lax.broadcasted_iota(jnp.int32, (B, S), 1)
    did = jnp.where(col == S - 1, 16, did)
    c_ref[...] = first * 32 + did


def _prep(input_ids, token_table, dinuc_table, proj_w, proj_b, ln_gamma,
          ln_beta):
    return pl.pallas_call(
        _prep_body,
        out_shape=(
            jax.ShapeDtypeStruct((8, 32, D), jnp.float32),
            jax.ShapeDtypeStruct((B, S), jnp.int32),
        ),
    )(input_ids, token_table, dinuc_table, proj_w,
      proj_b.reshape(1, D), ln_gamma.reshape(1, D), ln_beta.reshape(1, D))


PW = D // 2
LPITCH = PW + 1
NROW = 8 * 32
TT = TOK // NW
GT = 16
NG = TT // GT
OPITCH = D + 1
NCHAIN = 4


def _fill_block(idx_v, lut_v, ob, g, phase, iota16):
    c_vec = idx_v[pl.ds(g * GT, GT)]
    row_vec = phase * GT + iota16
    one = jnp.full((GT,), 1, jnp.int32)
    mask_hi = jnp.full((GT,), -65536, jnp.int32)
    la = [jnp.full((GT,), b, jnp.int32) for b in range(NCHAIN)]
    se = [jnp.full((GT,), 2 * b, jnp.int32) for b in range(NCHAIN)]
    dla = jnp.full((GT,), NCHAIN, jnp.int32)
    dse = jnp.full((GT,), 2 * NCHAIN, jnp.int32)
    for _ in range(PW // NCHAIN):
        for b in range(NCHAIN):
            w = plsc.load_gather(lut_v, [c_vec, la[b]])
            p0 = plsc.bitcast(w << 16, jnp.float32)
            p1 = plsc.bitcast(w & mask_hi, jnp.float32)
            plsc.store_scatter(ob, [row_vec, se[b]], p0)
            plsc.store_scatter(ob, [row_vec, se[b] + one], p1)
            la[b] = la[b] + dla
            se[b] = se[b] + dse


def _sc_gather_body(lut_hbm, idx_hbm, out_hbm, idx_v, lut_v, ob, sem0, sem1):
    wid = lax.axis_index("s") * NC + lax.axis_index("c")
    tok0 = wid * TT
    pltpu.sync_copy(lut_hbm, lut_v)
    pltpu.sync_copy(idx_hbm.at[pl.ds(tok0, TT)], idx_v)
    iota16 = lax.iota(jnp.int32, GT)

    def wcopy(i, phase, sem):
        return pltpu.make_async_copy(
            ob.at[pl.ds(phase * GT, GT), pl.ds(0, D)],
            out_hbm.at[pl.ds(tok0 + i * GT, GT)], sem)

    def body(i, carry):
        phase = lax.rem(i, 2)
        @pl.when(jnp.logical_and(i >= 2, phase == 0))
        def _():
            wcopy(i - 2, 0, sem0).wait()
        @pl.when(jnp.logical_and(i >= 2, phase == 1))
        def _():
            wcopy(i - 2, 1, sem1).wait()
        _fill_block(idx_v, lut_v, ob, i, phase, iota16)
        @pl.when(phase == 0)
        def _():
            wcopy(i, 0, sem0).start()
        @pl.when(phase == 1)
        def _():
            wcopy(i, 1, sem1).start()
        return carry

    lax.fori_loop(0, NG, body, 0)
    wcopy(NG - 2, 0, sem0).wait()
    wcopy(NG - 1, 1, sem1).wait()


@functools.cache
def _sc_gather():
    return pl.kernel(
        _sc_gather_body,
        out_type=jax.ShapeDtypeStruct((TOK, D), jnp.float32),
        mesh=plsc.VectorSubcoreMesh(core_axis_name="c", subcore_axis_name="s",
                                    num_cores=NC, num_subcores=NS),
        scratch_types=[
            pltpu.VMEM((TT,), jnp.int32),
            pltpu.VMEM((NROW, LPITCH), jnp.int32),
            pltpu.VMEM((2 * GT, OPITCH), jnp.float32),
            pltpu.SemaphoreType.DMA,
            pltpu.SemaphoreType.DMA,
        ],
        compiler_params=pltpu.CompilerParams(use_tc_tiling_on_sc=False,
                                             needs_layout_passes=False),
    )


@jax.jit
def kernel(input_ids, token_table, dinuc_table, proj_w, proj_b, ln_gamma,
           ln_beta):
    lut, c = _prep(input_ids, token_table, dinuc_table, proj_w, proj_b,
                   ln_gamma, ln_beta)
    lut_pk = lax.bitcast_convert_type(
        lut.astype(jnp.bfloat16).reshape(NROW, PW, 2), jnp.int32)
    lut_pk = jnp.pad(lut_pk, ((0, 0), (0, LPITCH - PW)))
    out = _sc_gather()(lut_pk, c.reshape(TOK))
    return out.reshape(B, S, D)


# speedup vs baseline: 1.7905x; 1.7905x over previous
import functools

import jax
import jax.numpy as jnp
from jax import lax
from jax.experimental import pallas as pl
from jax.experimental.pallas import tpu as pltpu
from jax.experimental.pallas import tpu_sc as plsc

B, S, D = 4, 8192, 768
DINUC_DIM = D // 4
EPS = 1e-12
TOK = B * S
NC, NS = 2, 16
NW = NC * NS
BPW = TOK // NW
CH = 64
NCH = BPW // CH


def _prep_body(ids_ref, tt_ref, dt_ref, w_ref, b_ref, g_ref, be_ref,
               lut_ref, c_ref):
    w_top = w_ref[:D, :]
    w_bot = w_ref[D:, :]
    tf = jnp.dot(tt_ref[...], w_top, preferred_element_type=jnp.float32)
    df = jnp.dot(dt_ref[...], w_bot, preferred_element_type=jnp.float32)
    df32 = jnp.concatenate([df, jnp.zeros((16, D), jnp.float32)], axis=0)
    x = tf[:, None, :] + df32[None, :, :] + b_ref[...][None, :, :]
    mean = jnp.mean(x, axis=-1, keepdims=True)
    var = jnp.mean((x - mean) ** 2, axis=-1, keepdims=True)
    lut_ref[...] = ((x - mean) * lax.rsqrt(var + EPS)
                    * g_ref[...][None, :, :] + be_ref[...][None, :, :])

    first = ids_ref[...]
    second = jnp.concatenate(
        [first[:, 1:], jnp.zeros((B, 1), jnp.int32)], axis=1)
    valid = ((first >= 4) & (first <= 7) & (second >= 4) & (second <= 7))
    did = jnp.where(valid, (first - 4) * 4 + (second - 4), 0)
    col = lax.broadcasted_iota(jnp.int32, (B, S), 1)
    did = jnp.where(col == S - 1, 16, did)
    c_ref[...] = first * 32 + did


def _prep(input_ids, token_table, dinuc_table, proj_w, proj_b, ln_gamma,
          ln_beta):
    return pl.pallas_call(
        _prep_body,
        out_shape=(
            jax.ShapeDtypeStruct((8, 32, D), jnp.float32),
            jax.ShapeDtypeStruct((B, S), jnp.int32),
        ),
    )(input_ids, token_table, dinuc_table, proj_w,
      proj_b.reshape(1, D), ln_gamma.reshape(1, D), ln_beta.reshape(1, D))


PW = D // 2
NROW = 8 * 32
TT = TOK // NW
GT = 16
NG = TT // GT


def _sc_gather_body(lut_hbm, idx_hbm, out_hbm, idx_v, lut_v, ob,
                    sem0, sem1):
    wid = lax.axis_index("s") * NC + lax.axis_index("c")
    tok0 = wid * TT
    pltpu.sync_copy(lut_hbm, lut_v)
    pltpu.sync_copy(idx_hbm.at[pl.ds(tok0, TT)], idx_v)
    mask_hi = jnp.full((GT,), -65536, jnp.int32)

    def wcopy(i, phase, sem):
        return pltpu.make_async_copy(
            ob.at[pl.ds(phase * GT * D, GT * D)],
            out_hbm.at[pl.ds((tok0 + i * GT) * D, GT * D)], sem)

    def body(i, carry):
        phase = lax.rem(i, 2)
        @pl.when(jnp.logical_and(i >= 2, phase == 0))
        def _():
            wcopy(i - 2, 0, sem0).wait()
        @pl.when(jnp.logical_and(i >= 2, phase == 1))
        def _():
            wcopy(i - 2, 1, sem1).wait()
        base_out = phase * (GT * D)
        cs = idx_v[pl.ds(i * GT, GT)]
        for t in range(GT):
            c = cs[t]
            src = c * PW
            dst = base_out + t * D
            for j in range(PW // GT):
                w = lut_v[pl.ds(src + GT * j, GT)]
                ob[pl.ds(dst + GT * j, GT)] = plsc.bitcast(
                    w << 16, jnp.float32)
                ob[pl.ds(dst + PW + GT * j, GT)] = plsc.bitcast(
                    w & mask_hi, jnp.float32)
        @pl.when(phase == 0)
        def _():
            wcopy(i, 0, sem0).start()
        @pl.when(phase == 1)
        def _():
            wcopy(i, 1, sem1).start()
        return carry

    lax.fori_loop(0, NG, body, 0)
    wcopy(NG - 2, 0, sem0).wait()
    wcopy(NG - 1, 1, sem1).wait()


@functools.cache
def _sc_gather():
    return pl.kernel(
        _sc_gather_body,
        out_type=jax.ShapeDtypeStruct((TOK * D,), jnp.float32),
        mesh=plsc.VectorSubcoreMesh(core_axis_name="c", subcore_axis_name="s",
                                    num_cores=NC, num_subcores=NS),
        scratch_types=[
            pltpu.VMEM((TT,), jnp.int32),
            pltpu.VMEM((NROW * PW,), jnp.int32),
            pltpu.VMEM((2 * GT * D,), jnp.float32),
            pltpu.SemaphoreType.DMA,
            pltpu.SemaphoreType.DMA,
        ],
        compiler_params=pltpu.CompilerParams(use_tc_tiling_on_sc=False,
                                             needs_layout_passes=False),
    )


@jax.jit
def kernel(input_ids, token_table, dinuc_table, proj_w, proj_b, ln_gamma,
           ln_beta):
    lut, c = _prep(input_ids, token_table, dinuc_table, proj_w, proj_b,
                   ln_gamma, ln_beta)
    lut_bf = lut.astype(jnp.bfloat16).reshape(NROW, D)
    pairs = jnp.stack([lut_bf[:, :PW], lut_bf[:, PW:]], axis=-1)
    lut_pk = lax.bitcast_convert_type(pairs, jnp.int32).reshape(NROW * PW)
    out = _sc_gather()(lut_pk, c.reshape(TOK))
    return out.reshape(B, S, D)


# speedup vs baseline: 2.9361x; 1.6398x over previous
import functools

import jax
import jax.numpy as jnp
from jax import lax
from jax.experimental import pallas as pl
from jax.experimental.pallas import tpu as pltpu
from jax.experimental.pallas import tpu_sc as plsc

B, S, D = 4, 8192, 768
DINUC_DIM = D // 4
EPS = 1e-12
TOK = B * S
NC, NS = 2, 16
NW = NC * NS
BPW = TOK // NW
CH = 64
NCH = BPW // CH


def _prep_body(ids_ref, tt_ref, dt_ref, w_ref, b_ref, g_ref, be_ref,
               lut_ref, c_ref):
    w_top = w_ref[:D, :]
    w_bot = w_ref[D:, :]
    tf = jnp.dot(tt_ref[...], w_top, preferred_element_type=jnp.float32)
    df = jnp.dot(dt_ref[...], w_bot, preferred_element_type=jnp.float32)
    df32 = jnp.concatenate([df, jnp.zeros((16, D), jnp.float32)], axis=0)
    x = tf[:, None, :] + df32[None, :, :] + b_ref[...][None, :, :]
    mean = jnp.mean(x, axis=-1, keepdims=True)
    var = jnp.mean((x - mean) ** 2, axis=-1, keepdims=True)
    lut_ref[...] = ((x - mean) * lax.rsqrt(var + EPS)
                    * g_ref[...][None, :, :] + be_ref[...][None, :, :])

    first = ids_ref[...]
    second = jnp.concatenate(
        [first[:, 1:], jnp.zeros((B, 1), jnp.int32)], axis=1)
    valid = ((first >= 4) & (first <= 7) & (second >= 4) & (second <= 7))
    did = jnp.where(valid, (first - 4) * 4 + (second - 4), 0)
    col = lax.broadcasted_iota(jnp.int32, (B, S), 1)
    did = jnp.where(col == S - 1, 16, did)
    c_ref[...] = first * 32 + did


def _prep(input_ids, token_table, dinuc_table, proj_w, proj_b, ln_gamma,
          ln_beta):
    return pl.pallas_call(
        _prep_body,
        out_shape=(
            jax.ShapeDtypeStruct((8, 32, D), jnp.float32),
            jax.ShapeDtypeStruct((B, S), jnp.int32),
        ),
    )(input_ids, token_table, dinuc_table, proj_w,
      proj_b.reshape(1, D), ln_gamma.reshape(1, D), ln_beta.reshape(1, D))


PW = D // 2
NROW = 8 * 32
TOK_SC = 8192
TOK_TC = TOK - TOK_SC
TT = TOK_SC // NW
GT = 16
NG = TT // GT
TBLK = 2048


def _sc_gather_body(lut_hbm, idx_hbm, out_hbm, idx_v, lut_v, ob,
                    sem0, sem1):
    wid = lax.axis_index("s") * NC + lax.axis_index("c")
    tok0 = wid * TT
    pltpu.sync_copy(lut_hbm, lut_v)
    pltpu.sync_copy(idx_hbm.at[pl.ds(tok0, TT)], idx_v)
    mask_hi = jnp.full((GT,), -65536, jnp.int32)

    def wcopy(i, phase, sem):
        return pltpu.make_async_copy(
            ob.at[pl.ds(phase * GT * D, GT * D)],
            out_hbm.at[pl.ds((tok0 + i * GT) * D, GT * D)], sem)

    def body(i, carry):
        phase = lax.rem(i, 2)
        @pl.when(jnp.logical_and(i >= 2, phase == 0))
        def _():
            wcopy(i - 2, 0, sem0).wait()
        @pl.when(jnp.logical_and(i >= 2, phase == 1))
        def _():
            wcopy(i - 2, 1, sem1).wait()
        base_out = phase * (GT * D)
        cs = idx_v[pl.ds(i * GT, GT)]
        for t in range(GT):
            c = cs[t]
            src = c * PW
            dst = base_out + t * D
            for j in range(PW // GT):
                w = lut_v[pl.ds(src + GT * j, GT)]
                ob[pl.ds(dst + GT * j, GT)] = plsc.bitcast(
                    w << 16, jnp.float32)
                ob[pl.ds(dst + PW + GT * j, GT)] = plsc.bitcast(
                    w & mask_hi, jnp.float32)
        @pl.when(phase == 0)
        def _():
            wcopy(i, 0, sem0).start()
        @pl.when(phase == 1)
        def _():
            wcopy(i, 1, sem1).start()
        return carry

    lax.fori_loop(0, NG, body, 0)
    wcopy(NG - 2, 0, sem0).wait()
    wcopy(NG - 1, 1, sem1).wait()


@functools.cache
def _sc_gather():
    return pl.kernel(
        _sc_gather_body,
        out_type=jax.ShapeDtypeStruct((TOK_SC * D,), jnp.float32),
        mesh=plsc.VectorSubcoreMesh(core_axis_name="c", subcore_axis_name="s",
                                    num_cores=NC, num_subcores=NS),
        scratch_types=[
            pltpu.VMEM((TT,), jnp.int32),
            pltpu.VMEM((NROW * PW,), jnp.int32),
            pltpu.VMEM((2 * GT * D,), jnp.float32),
            pltpu.SemaphoreType.DMA,
            pltpu.SemaphoreType.DMA,
        ],
        compiler_params=pltpu.CompilerParams(use_tc_tiling_on_sc=False,
                                             needs_layout_passes=False),
    )


def _tc_gather_body(c_ref, lut_ref, o_ref):
    onehot = (c_ref[...]
              == lax.broadcasted_iota(jnp.int32, (TBLK, NROW), 1)
              ).astype(jnp.bfloat16)
    o_ref[...] = jnp.dot(onehot, lut_ref[...],
                         preferred_element_type=jnp.float32)


def _tc_gather(c_tc, lut_bf):
    return pl.pallas_call(
        _tc_gather_body,
        grid=(TOK_TC // TBLK,),
        in_specs=[
            pl.BlockSpec((TBLK, 1), lambda i: (i, 0)),
            pl.BlockSpec((NROW, D), lambda i: (0, 0)),
        ],
        out_specs=pl.BlockSpec((TBLK, D), lambda i: (i, 0)),
        out_shape=jax.ShapeDtypeStruct((TOK_TC, D), jnp.float32),
    )(c_tc.reshape(-1, 1), lut_bf)


@jax.jit
def kernel(input_ids, token_table, dinuc_table, proj_w, proj_b, ln_gamma,
           ln_beta):
    lut, c = _prep(input_ids, token_table, dinuc_table, proj_w, proj_b,
                   ln_gamma, ln_beta)
    lut_bf = lut.astype(jnp.bfloat16).reshape(NROW, D)
    pairs = jnp.stack([lut_bf[:, :PW], lut_bf[:, PW:]], axis=-1)
    lut_pk = lax.bitcast_convert_type(pairs, jnp.int32).reshape(NROW * PW)
    cf = c.reshape(TOK)
    out_sc = _sc_gather()(lut_pk, cf[:TOK_SC]).reshape(TOK_SC, D)
    out_tc = _tc_gather(cf[TOK_SC:], lut_bf)
    return jnp.concatenate([out_sc, out_tc], axis=0).reshape(B, S, D)
